# Initial kernel scaffold; baseline (speedup 1.0000x reference)
import jax
import jax.numpy as jnp
from jax.experimental import pallas as pl


def kernel(x_1, edge_index_1, edge_attr_1, batch_1, T, ratio_1, h_inter, h_intra_1, x_2, edge_index_2, edge_attr_2, batch_2, ratio_2, W_lin0, b_lin0, W_tr, b_tr, W_g1, b_g1, W_g2, b_g2, W_mtr, b_mtr, W_m1, b_m1, W_m2, b_m2, W_fc1, b_fc1, W_fc2, b_fc2, W_fc3, b_fc3):
    raise NotImplementedError("write your pallas kernel here")


# speedup vs baseline: 2.9812x; 2.9812x over previous
import functools

import jax
import jax.numpy as jnp
from jax import lax
from jax.experimental import pallas as pl
from jax.experimental.pallas import tpu as pltpu
from jax.experimental.pallas import tpu_sc as plsc

DIM = 128
NF = 128
NEF = 16
N = 10000
E = 320000
B = 256

_RB = 1000


def _lin0_body(x_ref, w_ref, b_ref, o_ref):
    y = jnp.dot(x_ref[...], w_ref[...], preferred_element_type=jnp.float32)
    o_ref[...] = jnp.maximum(y + b_ref[...][None, :], 0.0)


def _lin0(x_cat, W, b):
    nb = (2 * N) // _RB
    return pl.pallas_call(
        _lin0_body,
        grid=(nb,),
        in_specs=[
            pl.BlockSpec((_RB, NF), lambda i: (i, 0)),
            pl.BlockSpec((NF, DIM), lambda i: (0, 0)),
            pl.BlockSpec((DIM,), lambda i: (0,)),
        ],
        out_specs=pl.BlockSpec((_RB, DIM), lambda i: (i, 0)),
        out_shape=jax.ShapeDtypeStruct((2 * N, DIM), jnp.float32),
    )(x_cat, W, b)


_REB = 4000


def _eatr_body(a_ref, w_ref, b_ref, o_ref):
    y = jnp.dot(a_ref[...], w_ref[...], preferred_element_type=jnp.float32)
    o_ref[...] = y + b_ref[...][None, :]


def _eatr(ea_cat, W, b):
    nb = (2 * E) // _REB
    return pl.pallas_call(
        _eatr_body,
        grid=(nb,),
        in_specs=[
            pl.BlockSpec((_REB, NEF), lambda i: (i, 0)),
            pl.BlockSpec((NEF, DIM), lambda i: (0, 0)),
            pl.BlockSpec((DIM,), lambda i: (0,)),
        ],
        out_specs=pl.BlockSpec((_REB, DIM), lambda i: (i, 0)),
        out_shape=jax.ShapeDtypeStruct((2 * E, DIM), jnp.float32),
    )(ea_cat, W, b)


_CH = 64
_NSUB = 16
_EPT = E // _NSUB
_NCHUNK = -(-_EPT // _CH) + 0
_CPS = _NCHUNK + 1
_NA = N + 8
_RPS = 624
_ZR = 208
_ZTAIL = _NA - _NSUB * _RPS
_OTAIL = N - _NSUB * _RPS


def _edge_body(x_hbm, ea_hbm, ei_hbm, z_hbm, out_hbm,
               idx0, idx1, ea0, ea1, xr0, xr1, agg_sh,
               sem_i0, sem_i1, sem_e0, sem_e1, sem_g0, sem_g1):
    c = lax.axis_index("c")
    s = lax.axis_index("s")
    idx = (idx0, idx1)
    ea = (ea0, ea1)
    xr = (xr0, xr1)
    sem_i = (sem_i0, sem_i1)
    sem_e = (sem_e0, sem_e1)
    sem_g = (sem_g0, sem_g1)

    def zero_body(i, carry):
        r0 = s * _RPS + i * _ZR
        pltpu.sync_copy(z_hbm.at[pl.ds(r0, _ZR)], agg_sh.at[pl.ds(r0, _ZR)])
        return carry

    lax.fori_loop(0, _RPS // _ZR, zero_body, 0)

    @pl.when(s == _NSUB - 1)
    def _():
        pltpu.sync_copy(z_hbm.at[pl.ds(_NSUB * _RPS, _ZTAIL)],
                        agg_sh.at[pl.ds(_NSUB * _RPS, _ZTAIL)])

    plsc.subcore_barrier()

    cbase = (c * _NSUB + s) * _CPS
    ea_off = c * E + s * _EPT

    def issue_idx(g, b):
        pltpu.async_copy(ei_hbm.at[cbase + g], idx[b], sem_i[b])

    def wait_idx(b):
        pltpu.make_async_copy(ei_hbm.at[0], idx[b], sem_i[b]).wait()

    def issue_data(g, b):
        eoff = ea_off + lax.min(g * _CH, _EPT - _CH)
        pltpu.async_copy(ea_hbm.at[pl.ds(eoff, _CH)],
                         ea[b], sem_e[b])
        pltpu.async_copy(x_hbm.at[idx[b].at[0]], xr[b], sem_g[b])

    def wait_data(b):
        pltpu.make_async_copy(ea_hbm.at[pl.ds(0, _CH)], ea[b], sem_e[b]).wait()
        pltpu.make_async_copy(x_hbm.at[pl.ds(0, _CH)], xr[b], sem_g[b]).wait()

    def compute(b):
        eb, xb = ea[b], xr[b]

        def row(r, rc):
            for k in range(DIM // 16):
                sl = pl.ds(k * 16, 16)
                eb[r, sl] = jnp.maximum(eb[r, sl] + xb[r, sl], 0.0)
            return rc

        lax.fori_loop(0, _CH, row, 0)

    def scatter(b):
        pltpu.sync_copy(ea[b], agg_sh.at[idx[b].at[1]], add=True)

    issue_idx(0, 0)
    issue_idx(1, 1)
    wait_idx(0)
    issue_data(0, 0)

    def body(g, b):
        nb = 1 - b
        wait_idx(nb)
        issue_data(g + 1, nb)
        wait_data(b)
        compute(b)
        scatter(b)
        issue_idx(g + 2, b)

    def loop2(i, carry):
        body(2 * i, 0)
        body(2 * i + 1, 1)
        return carry

    lax.fori_loop(0, (_NCHUNK - 1) // 2, loop2, 0)

    wait_data(0)
    compute(0)
    scatter(0)
    wait_idx(1)

    plsc.subcore_barrier()

    def out_body(i, carry):
        r0 = s * _RPS + i * _ZR
        pltpu.sync_copy(agg_sh.at[pl.ds(r0, _ZR)],
                        out_hbm.at[pl.ds(c * N + r0, _ZR)])
        return carry

    lax.fori_loop(0, _RPS // _ZR, out_body, 0)

    @pl.when(s == _NSUB - 1)
    def _():
        pltpu.sync_copy(agg_sh.at[pl.ds(_NSUB * _RPS, _OTAIL)],
                        out_hbm.at[pl.ds(c * N + _NSUB * _RPS, _OTAIL)])


@functools.cache
def _edge_pass_fn():
    return pl.kernel(
        _edge_body,
        out_type=jax.ShapeDtypeStruct((2 * N, DIM), jnp.float32),
        mesh=plsc.VectorSubcoreMesh(core_axis_name="c", subcore_axis_name="s"),
        scratch_types=[
            pltpu.VMEM((2, _CH), jnp.int32),
            pltpu.VMEM((2, _CH), jnp.int32),
            pltpu.VMEM((_CH, DIM), jnp.float32),
            pltpu.VMEM((_CH, DIM), jnp.float32),
            pltpu.VMEM((_CH, DIM), jnp.float32),
            pltpu.VMEM((_CH, DIM), jnp.float32),
            pltpu.VMEM_SHARED((_NA, DIM), jnp.float32),
            pltpu.SemaphoreType.DMA,
            pltpu.SemaphoreType.DMA,
            pltpu.SemaphoreType.DMA,
            pltpu.SemaphoreType.DMA,
            pltpu.SemaphoreType.DMA,
            pltpu.SemaphoreType.DMA,
        ],
    )


def _pack_indices(edge_index_1, edge_index_2):
    pads = _NCHUNK * _CH - _EPT
    blocks = []
    for bi, ei in ((0, edge_index_1), (1, edge_index_2)):
        src = ei[0].reshape(_NSUB, _EPT) + bi * N
        dst = ei[1].reshape(_NSUB, _EPT)
        src = jnp.concatenate(
            [src, jnp.full((_NSUB, pads), bi * N, jnp.int32)], axis=1)
        dst = jnp.concatenate(
            [dst, jnp.full((_NSUB, pads), N, jnp.int32)], axis=1)
        sd = jnp.stack([src.reshape(_NSUB, _NCHUNK, _CH),
                        dst.reshape(_NSUB, _NCHUNK, _CH)], axis=2)
        sd = jnp.concatenate(
            [sd, jnp.zeros((_NSUB, 1, 2, _CH), jnp.int32)], axis=1)
        blocks.append(sd.reshape(_NSUB * _CPS, 2, _CH))
    return jnp.concatenate(blocks, axis=0)


def _edge_pass(x_flat, ea_flat, ei_packed, zeros):
    return _edge_pass_fn()(x_flat, ea_flat, ei_packed, zeros)


def _gine_pool_body(x_ref, agg_ref, ids_ref, w1_ref, b1_ref, w2_ref, b2_ref,
                    o_ref):
    i = pl.program_id(1)
    h = x_ref[...] + agg_ref[...]
    t = jnp.dot(h, w1_ref[...], preferred_element_type=jnp.float32)
    t = jnp.maximum(t + b1_ref[...][None, :], 0.0)
    y = jnp.dot(t, w2_ref[...], preferred_element_type=jnp.float32)
    y = jnp.maximum(y + b2_ref[...][None, :], 0.0)
    ids = ids_ref[0, 0, :]
    oh = (lax.broadcasted_iota(jnp.int32, (B, _RB), 0) == ids[None, :])
    part = jnp.dot(oh.astype(jnp.float32), y,
                   preferred_element_type=jnp.float32,
                   precision=lax.Precision.HIGHEST)

    @pl.when(i == 0)
    def _():
        o_ref[...] = jnp.zeros_like(o_ref)

    o_ref[0] += part


def _gine_pool(x_flat, agg_flat, ids3, W1, b1, W2, b2):
    nb = N // _RB
    return pl.pallas_call(
        _gine_pool_body,
        grid=(2, nb),
        in_specs=[
            pl.BlockSpec((_RB, DIM), lambda g, i: (g * (N // _RB) + i, 0)),
            pl.BlockSpec((_RB, DIM), lambda g, i: (g * (N // _RB) + i, 0)),
            pl.BlockSpec((1, 1, _RB), lambda g, i: (g * (N // _RB) + i, 0, 0)),
            pl.BlockSpec((DIM, 2 * DIM), lambda g, i: (0, 0)),
            pl.BlockSpec((2 * DIM,), lambda g, i: (0,)),
            pl.BlockSpec((2 * DIM, DIM), lambda g, i: (0, 0)),
            pl.BlockSpec((DIM,), lambda g, i: (0,)),
        ],
        out_specs=pl.BlockSpec((1, B, DIM), lambda g, i: (g, 0, 0)),
        out_shape=jax.ShapeDtypeStruct((2, B, DIM), jnp.float32),
    )(x_flat, agg_flat, ids3, W1, b1, W2, b2)


def _head_body(p_ref, r1_ref, r2_ref, hint_ref, hintra_ref, t10_ref,
               wmtr_ref, bmtr_ref, wm1_ref, bm1_ref, wm2_ref, bm2_ref,
               wf1_ref, bf1_ref, wf2a_ref, wf2b_ref, bf2_ref,
               wf3_ref, bf3_ref, o_ref):
    x1 = p_ref[0] * r1_ref[...]
    x2 = p_ref[1] * r2_ref[...]
    rz = r2_ref[...] == 0.0
    node2 = jnp.where(rz, x1, x2)
    evl = jnp.where(rz, hintra_ref[...], hint_ref[...])
    bf = lambda v: v.astype(jnp.bfloat16).astype(jnp.float32)
    ea = bf(evl) * bf(wmtr_ref[...]) + bmtr_ref[...][None, :]

    he = x1 + jnp.maximum(node2 + ea, 0.0)
    ho = node2 + jnp.maximum(x1 + ea, 0.0)

    def mlp(h):
        t = jnp.dot(h, wm1_ref[...], preferred_element_type=jnp.float32)
        t = jnp.maximum(t + bm1_ref[...][None, :], 0.0)
        y = jnp.dot(t, wm2_ref[...], preferred_element_type=jnp.float32)
        return jnp.maximum(y + bm2_ref[...][None, :], 0.0)

    fp = mlp(he) + mlp(ho)
    t = jnp.dot(fp, wf1_ref[...], preferred_element_type=jnp.float32)
    t = jnp.maximum(t + bf1_ref[...][None, :], 0.0)
    u = jnp.dot(t, wf2a_ref[...], preferred_element_type=jnp.float32)
    u = u + t10_ref[...] * wf2b_ref[...]
    u = jnp.maximum(u + bf2_ref[...][None, :], 0.0)
    o_ref[...] = (jnp.dot(u, wf3_ref[...], preferred_element_type=jnp.float32)
                  + bf3_ref[...])


def _head(pooled, r1, r2, hint, hintra, t10,
          W_mtr, b_mtr, W_m1, b_m1, W_m2, b_m2,
          W_fc1, b_fc1, W_fc2a, w_fc2b, b_fc2, W_fc3, b_fc3):
    return pl.pallas_call(
        _head_body,
        out_shape=jax.ShapeDtypeStruct((B, 1), jnp.float32),
    )(pooled, r1, r2, hint, hintra, t10,
      W_mtr, b_mtr, W_m1, b_m1, W_m2, b_m2,
      W_fc1, b_fc1, W_fc2a, w_fc2b, b_fc2, W_fc3, b_fc3)


def kernel(x_1, edge_index_1, edge_attr_1, batch_1, T, ratio_1, h_inter,
           h_intra_1, x_2, edge_index_2, edge_attr_2, batch_2, ratio_2,
           W_lin0, b_lin0, W_tr, b_tr, W_g1, b_g1, W_g2, b_g2,
           W_mtr, b_mtr, W_m1, b_m1, W_m2, b_m2,
           W_fc1, b_fc1, W_fc2, b_fc2, W_fc3, b_fc3):
    x_cat = jnp.concatenate([x_1, x_2], axis=0)
    ea_cat = jnp.concatenate([edge_attr_1, edge_attr_2], axis=0)
    ei_packed = _pack_indices(edge_index_1, edge_index_2)
    zeros = jnp.zeros((_NA, DIM), jnp.float32)
    ids3 = jnp.concatenate([batch_1, batch_2]).reshape(2 * N // _RB, 1, _RB)

    x_flat = _lin0(x_cat, W_lin0, b_lin0)
    ea_flat = _eatr(ea_cat, W_tr, b_tr)
    agg_flat = _edge_pass(x_flat, ea_flat, ei_packed, zeros)
    pooled = _gine_pool(x_flat, agg_flat, ids3, W_g1, b_g1, W_g2, b_g2)

    col = lambda v: v.reshape(B, 1).astype(jnp.float32)
    out = _head(pooled, col(ratio_1), col(ratio_2), col(h_inter),
                col(h_intra_1), col(10.0 * T),
                W_mtr, b_mtr, W_m1, b_m1, W_m2, b_m2,
                W_fc1, b_fc1, W_fc2[:DIM], W_fc2[DIM:], b_fc2,
                W_fc3, b_fc3.reshape(1, 1))
    return out
